# baseline (speedup 1.0000x reference)
import functools

import jax
import jax.numpy as jnp
from jax import lax
from jax.experimental import pallas as pl
from jax.experimental.pallas import tpu as pltpu
from jax.experimental.pallas import tpu_sc as plsc

N = 10000
E = 320000
D = 128
DP = 144
NC = 2
NS = 16
NW = NC * NS
C = 80
NCHUNK = E // (NW * C)
ROWS_PER_TILE = N // NS
ZROWS = 25
OROWS = 125
IH = 63

_mesh = plsc.VectorSubcoreMesh(
    core_axis_name="c", subcore_axis_name="s", num_cores=NC, num_subcores=NS
)


@functools.partial(
    pl.kernel,
    out_type=jax.ShapeDtypeStruct((NC, N, DP), jnp.float32),
    mesh=_mesh,
    scratch_types=[
        pltpu.VMEM((IH, C), jnp.int32),
        pltpu.VMEM((IH, C), jnp.int32),
        pltpu.VMEM((2, C, DP), jnp.float32),
        pltpu.VMEM((ZROWS, DP), jnp.float32),
        pltpu.VMEM_SHARED((N, DP), jnp.float32),
        pltpu.SemaphoreType.DMA((2,)),
        pltpu.SemaphoreType.DMA((2,)),
    ],
    compiler_params=pltpu.CompilerParams(use_tc_tiling_on_sc=False),
)
def _sc_aggregate(xp_hbm, src_hbm, dst_hbm, out_hbm,
                  src_v, dst_v, rows_v, zbuf, acc, sem, sem_s):
    cid = lax.axis_index("c")
    sid = lax.axis_index("s")
    wid = cid * NS + sid

    def _zero_row(i, _):
        r = i // (DP // 16)
        f = i % (DP // 16)
        zbuf[r, pl.ds(f * 16, 16)] = jnp.zeros((16,), jnp.float32)
        return 0
    lax.fori_loop(0, ZROWS * (DP // 16), _zero_row, 0)
    for kz in range(ROWS_PER_TILE // ZROWS):
        pltpu.sync_copy(zbuf, acc.at[pl.ds(sid * ROWS_PER_TILE + kz * ZROWS, ZROWS)])
    plsc.subcore_barrier()

    pltpu.sync_copy(src_hbm.at[wid].at[pl.ds(0, IH)], src_v)
    pltpu.sync_copy(dst_hbm.at[wid].at[pl.ds(0, IH)], dst_v)

    def _gather(j, buf):
        row = jnp.where(j < IH, j, j - IH)
        pltpu.async_copy(xp_hbm.at[src_v.at[row]], rows_v.at[buf], sem.at[buf])

    def _gather_wait(buf):
        pltpu.make_async_copy(
            xp_hbm.at[src_v.at[0]], rows_v.at[buf], sem.at[buf]).wait()

    _gather(0, 0)

    def _scatter_wait(buf):
        pltpu.make_async_copy(
            rows_v.at[buf], acc.at[dst_v.at[0]], sem_s.at[buf]).wait()

    def _edge_chunk(j, _):
        buf = lax.rem(j, 2)
        nbuf = 1 - buf
        _gather_wait(buf)

        @pl.when(j == IH - 1)
        def _():
            pltpu.sync_copy(src_hbm.at[wid].at[pl.ds(IH, NCHUNK - IH)],
                            src_v.at[pl.ds(0, NCHUNK - IH)])

        @pl.when(j >= 1)
        def _():
            _scatter_wait(nbuf)

        @pl.when(j == IH)
        def _():
            pltpu.sync_copy(dst_hbm.at[wid].at[pl.ds(IH, NCHUNK - IH)],
                            dst_v.at[pl.ds(0, NCHUNK - IH)])

        @pl.when(j + 1 < NCHUNK)
        def _():
            _gather(j + 1, nbuf)

        row = jnp.where(j < IH, j, j - IH)
        pltpu.async_copy(rows_v.at[buf], acc.at[dst_v.at[row]],
                         sem_s.at[buf], add=True)
        return 0
    lax.fori_loop(0, NCHUNK, _edge_chunk, 0)
    _scatter_wait((NCHUNK - 1) % 2)

    plsc.subcore_barrier()
    for kz in range(ROWS_PER_TILE // OROWS):
        r0 = sid * ROWS_PER_TILE + kz * OROWS
        pltpu.sync_copy(acc.at[pl.ds(r0, OROWS)], out_hbm.at[cid].at[pl.ds(r0, OROWS)])


def _tc_finish(p_ref, x_ref, w_ref, b_ref, o_ref):
    p = p_ref[0] + p_ref[1]
    agg = p[:, :D]
    deg = jnp.maximum(p[:, D:D + 1], 1.0)
    h = agg / deg + x_ref[...]
    y = jnp.dot(h, w_ref[...], preferred_element_type=jnp.float32) + b_ref[...]
    o_ref[...] = jnp.maximum(y, 0.0)


def kernel(x, edge_index, W, b):
    ei = edge_index.astype(jnp.int32)
    src2d = ei[0].reshape(NW, NCHUNK, C)
    dst2d = ei[1].reshape(NW, NCHUNK, C)
    xp = jnp.concatenate(
        [x, jnp.ones((N, 1), x.dtype), jnp.zeros((N, DP - D - 1), x.dtype)], axis=1
    )
    partials = _sc_aggregate(xp, src2d, dst2d)

    BN = 1000
    out = pl.pallas_call(
        _tc_finish,
        grid=(N // BN,),
        in_specs=[
            pl.BlockSpec((NC, BN, DP), lambda i: (0, i, 0)),
            pl.BlockSpec((BN, D), lambda i: (i, 0)),
            pl.BlockSpec((D, D), lambda i: (0, 0)),
            pl.BlockSpec((1, D), lambda i: (0, 0)),
        ],
        out_specs=pl.BlockSpec((BN, D), lambda i: (i, 0)),
        out_shape=jax.ShapeDtypeStruct((N, D), jnp.float32),
    )(partials, x, W, b.reshape(1, D))
    return out


# speedup vs baseline: 1.1659x; 1.1659x over previous
import functools

import jax
import jax.numpy as jnp
from jax import lax
from jax.experimental import pallas as pl
from jax.experimental.pallas import tpu as pltpu
from jax.experimental.pallas import tpu_sc as plsc

N = 10000
E = 320000
D = 128
DG = 16
NC = 2
NS = 16
NW = NC * NS
C = 80
NCHUNK = E // (NW * C)
ROWS_PER_TILE = N // NS
ZROWS = 25
OROWS = 125
IH = 63

_mesh = plsc.VectorSubcoreMesh(
    core_axis_name="c", subcore_axis_name="s", num_cores=NC, num_subcores=NS
)


@functools.partial(
    pl.kernel,
    out_type=(
        jax.ShapeDtypeStruct((NC, N, D), jnp.float32),
        jax.ShapeDtypeStruct((NC, N, DG), jnp.float32),
    ),
    mesh=_mesh,
    scratch_types=[
        pltpu.VMEM((IH, C), jnp.int32),
        pltpu.VMEM((IH, C), jnp.int32),
        pltpu.VMEM((2, C, D), jnp.float32),
        pltpu.VMEM((C, DG), jnp.float32),
        pltpu.VMEM((ZROWS, D), jnp.float32),
        pltpu.VMEM((ZROWS, DG), jnp.float32),
        pltpu.VMEM_SHARED((N, D), jnp.float32),
        pltpu.VMEM_SHARED((N, DG), jnp.float32),
        pltpu.SemaphoreType.DMA((2,)),
        pltpu.SemaphoreType.DMA((2,)),
        pltpu.SemaphoreType.DMA((2,)),
    ],
    compiler_params=pltpu.CompilerParams(use_tc_tiling_on_sc=False),
)
def _sc_aggregate(x_hbm, src_hbm, dst_hbm, out_hbm, deg_hbm,
                  src_v, dst_v, rows_v, ones_v, zbuf, zbuf_d, acc, dacc,
                  sem, sem_s, sem_d):
    cid = lax.axis_index("c")
    sid = lax.axis_index("s")
    wid = cid * NS + sid

    def _zero_row(i, _):
        r = i // (D // 16)
        f = i % (D // 16)
        zbuf[r, pl.ds(f * 16, 16)] = jnp.zeros((16,), jnp.float32)
        return 0
    lax.fori_loop(0, ZROWS * (D // 16), _zero_row, 0)

    def _zero_drow(r, _):
        zbuf_d[r, pl.ds(0, 16)] = jnp.zeros((16,), jnp.float32)
        return 0
    lax.fori_loop(0, ZROWS, _zero_drow, 0)

    def _one_row(r, _):
        ones_v[r, pl.ds(0, 16)] = jnp.ones((16,), jnp.float32)
        return 0
    lax.fori_loop(0, C, _one_row, 0)

    for kz in range(ROWS_PER_TILE // ZROWS):
        r0 = sid * ROWS_PER_TILE + kz * ZROWS
        pltpu.sync_copy(zbuf, acc.at[pl.ds(r0, ZROWS)])
        pltpu.sync_copy(zbuf_d, dacc.at[pl.ds(r0, ZROWS)])
    plsc.subcore_barrier()

    pltpu.sync_copy(src_hbm.at[wid].at[pl.ds(0, IH)], src_v)
    pltpu.sync_copy(dst_hbm.at[wid].at[pl.ds(0, IH)], dst_v)

    def _gather(j, buf):
        row = jnp.where(j < IH, j, j - IH)
        pltpu.async_copy(x_hbm.at[src_v.at[row]], rows_v.at[buf], sem.at[buf])

    def _gather_wait(buf):
        pltpu.make_async_copy(
            x_hbm.at[src_v.at[0]], rows_v.at[buf], sem.at[buf]).wait()

    def _scatter_wait(buf):
        pltpu.make_async_copy(
            rows_v.at[buf], acc.at[dst_v.at[0]], sem_s.at[buf]).wait()

    def _dscatter_wait(buf):
        pltpu.make_async_copy(
            ones_v, dacc.at[dst_v.at[0]], sem_d.at[buf]).wait()

    _gather(0, 0)

    def _edge_chunk(j, _):
        buf = lax.rem(j, 2)
        nbuf = 1 - buf
        _gather_wait(buf)

        @pl.when(j == IH - 1)
        def _():
            pltpu.sync_copy(src_hbm.at[wid].at[pl.ds(IH, NCHUNK - IH)],
                            src_v.at[pl.ds(0, NCHUNK - IH)])

        @pl.when(j >= 1)
        def _():
            _scatter_wait(nbuf)
            _dscatter_wait(nbuf)

        @pl.when(j == IH)
        def _():
            pltpu.sync_copy(dst_hbm.at[wid].at[pl.ds(IH, NCHUNK - IH)],
                            dst_v.at[pl.ds(0, NCHUNK - IH)])

        @pl.when(j + 1 < NCHUNK)
        def _():
            _gather(j + 1, nbuf)

        row = jnp.where(j < IH, j, j - IH)
        pltpu.async_copy(rows_v.at[buf], acc.at[dst_v.at[row]],
                         sem_s.at[buf], add=True)
        pltpu.async_copy(ones_v, dacc.at[dst_v.at[row]],
                         sem_d.at[buf], add=True)
        return 0
    lax.fori_loop(0, NCHUNK, _edge_chunk, 0)
    _scatter_wait((NCHUNK - 1) % 2)
    _dscatter_wait((NCHUNK - 1) % 2)

    plsc.subcore_barrier()
    for kz in range(ROWS_PER_TILE // OROWS):
        r0 = sid * ROWS_PER_TILE + kz * OROWS
        pltpu.sync_copy(acc.at[pl.ds(r0, OROWS)], out_hbm.at[cid].at[pl.ds(r0, OROWS)])
    pltpu.sync_copy(dacc.at[pl.ds(sid * ROWS_PER_TILE, ROWS_PER_TILE)],
                    deg_hbm.at[cid].at[pl.ds(sid * ROWS_PER_TILE, ROWS_PER_TILE)])


def _tc_finish(p_ref, d_ref, x_ref, w_ref, b_ref, o_ref):
    agg = p_ref[0] + p_ref[1]
    deg16 = d_ref[0] + d_ref[1]
    deg = jnp.maximum(deg16[:, 0:1], 1.0)
    h = agg / deg + x_ref[...]
    y = jnp.dot(h, w_ref[...], preferred_element_type=jnp.float32) + b_ref[...]
    o_ref[...] = jnp.maximum(y, 0.0)


def kernel(x, edge_index, W, b):
    ei = edge_index.astype(jnp.int32)
    src2d = ei[0].reshape(NW, NCHUNK, C)
    dst2d = ei[1].reshape(NW, NCHUNK, C)
    partials, degs = _sc_aggregate(x, src2d, dst2d)

    BN = 1000
    out = pl.pallas_call(
        _tc_finish,
        grid=(N // BN,),
        in_specs=[
            pl.BlockSpec((NC, BN, D), lambda i: (0, i, 0)),
            pl.BlockSpec((NC, BN, DG), lambda i: (0, i, 0)),
            pl.BlockSpec((BN, D), lambda i: (i, 0)),
            pl.BlockSpec((D, D), lambda i: (0, 0)),
            pl.BlockSpec((1, D), lambda i: (0, 0)),
        ],
        out_specs=pl.BlockSpec((BN, D), lambda i: (i, 0)),
        out_shape=jax.ShapeDtypeStruct((N, D), jnp.float32),
    )(partials, degs, x, W, b.reshape(1, D))
    return out


# speedup vs baseline: 1.5083x; 1.2937x over previous
import functools

import jax
import jax.numpy as jnp
from jax import lax
from jax.experimental import pallas as pl
from jax.experimental.pallas import tpu as pltpu
from jax.experimental.pallas import tpu_sc as plsc

N = 10000
E = 320000
D = 128
DG = 16
NC = 2
NS = 16
NW = NC * NS
C = 80
NCHUNK = E // (NW * C)
ROWS_PER_TILE = N // NS
OROWS = 125
IH = 63
QD = 32
NBUF = 3

_mesh = plsc.VectorSubcoreMesh(
    core_axis_name="c", subcore_axis_name="s", num_cores=NC, num_subcores=NS
)


@functools.partial(
    pl.kernel,
    out_type=(
        jax.ShapeDtypeStruct((NC, N, D), jnp.float32),
        jax.ShapeDtypeStruct((NC, N, DG), jnp.float32),
    ),
    mesh=_mesh,
    scratch_types=[
        pltpu.VMEM((IH, C), jnp.int32),
        pltpu.VMEM((QD, C), jnp.int32),
        pltpu.VMEM((NBUF, C, D), jnp.float32),
        pltpu.VMEM((C, DG), jnp.float32),
        pltpu.VMEM_SHARED((N, D), jnp.float32),
        pltpu.VMEM_SHARED((N, DG), jnp.float32),
        pltpu.SemaphoreType.DMA((NBUF,)),
        pltpu.SemaphoreType.DMA((NBUF,)),
        pltpu.SemaphoreType.DMA((NBUF,)),
    ],
    compiler_params=pltpu.CompilerParams(use_tc_tiling_on_sc=False),
)
def _sc_aggregate(x_hbm, src_hbm, dst_hbm, z128_hbm, z16_hbm, ones_hbm,
                  out_hbm, deg_hbm,
                  src_v, dst_v, rows_v, ones_v, acc, dacc,
                  sem, sem_s, sem_d):
    cid = lax.axis_index("c")
    sid = lax.axis_index("s")
    wid = cid * NS + sid

    for kz in range(ROWS_PER_TILE // OROWS):
        r0 = sid * ROWS_PER_TILE + kz * OROWS
        pltpu.sync_copy(z128_hbm, acc.at[pl.ds(r0, OROWS)])
        pltpu.sync_copy(z16_hbm, dacc.at[pl.ds(r0, OROWS)])
    pltpu.sync_copy(ones_hbm, ones_v)
    plsc.subcore_barrier()

    pltpu.sync_copy(src_hbm.at[wid].at[pl.ds(0, IH)], src_v)
    pltpu.sync_copy(dst_hbm.at[wid].at[pl.ds(0, QD)], dst_v)

    def _gather(j, buf):
        row = jnp.where(j < IH, j, j - IH)
        pltpu.async_copy(x_hbm.at[src_v.at[row]], rows_v.at[buf], sem.at[buf])

    def _gather_wait(buf):
        pltpu.make_async_copy(
            x_hbm.at[src_v.at[0]], rows_v.at[buf], sem.at[buf]).wait()

    def _scatter_wait(buf):
        pltpu.make_async_copy(
            rows_v.at[buf], acc.at[dst_v.at[0]], sem_s.at[buf]).wait()

    def _dscatter_wait(buf):
        pltpu.make_async_copy(
            ones_v, dacc.at[dst_v.at[0]], sem_d.at[buf]).wait()

    _gather(0, 0)
    _gather(1, 1)

    def _edge_chunk(j, _):
        buf = lax.rem(j, NBUF)
        _gather_wait(buf)

        @pl.when(j >= 1)
        def _():
            _scatter_wait(lax.rem(j - 1, NBUF))
            _dscatter_wait(lax.rem(j - 1, NBUF))

        @pl.when(j == IH - 2)
        def _():
            pltpu.sync_copy(src_hbm.at[wid].at[pl.ds(IH, NCHUNK - IH)],
                            src_v.at[pl.ds(0, NCHUNK - IH)])

        for jq, qn in ((QD, QD), (2 * QD, QD), (3 * QD, NCHUNK - 3 * QD)):
            @pl.when(j == jq)
            def _(jq=jq, qn=qn):
                pltpu.sync_copy(dst_hbm.at[wid].at[pl.ds(jq, qn)],
                                dst_v.at[pl.ds(0, qn)])

        @pl.when(j + 2 < NCHUNK)
        def _():
            _gather(j + 2, lax.rem(j + 2, NBUF))

        drow = lax.rem(j, QD)
        pltpu.async_copy(rows_v.at[buf], acc.at[dst_v.at[drow]],
                         sem_s.at[buf], add=True)
        pltpu.async_copy(ones_v, dacc.at[dst_v.at[drow]],
                         sem_d.at[buf], add=True)
        return 0
    lax.fori_loop(0, NCHUNK, _edge_chunk, 0)
    _scatter_wait((NCHUNK - 1) % NBUF)
    _dscatter_wait((NCHUNK - 1) % NBUF)

    plsc.subcore_barrier()
    for kz in range(ROWS_PER_TILE // OROWS):
        r0 = sid * ROWS_PER_TILE + kz * OROWS
        pltpu.sync_copy(acc.at[pl.ds(r0, OROWS)], out_hbm.at[cid].at[pl.ds(r0, OROWS)])
    pltpu.sync_copy(dacc.at[pl.ds(sid * ROWS_PER_TILE, ROWS_PER_TILE)],
                    deg_hbm.at[cid].at[pl.ds(sid * ROWS_PER_TILE, ROWS_PER_TILE)])


def _tc_finish(p_ref, d_ref, x_ref, w_ref, b_ref, o_ref):
    agg = p_ref[0] + p_ref[1]
    deg16 = d_ref[0] + d_ref[1]
    deg = jnp.maximum(deg16[:, 0:1], 1.0)
    h = agg / deg + x_ref[...]
    y = jnp.dot(h, w_ref[...], preferred_element_type=jnp.float32) + b_ref[...]
    o_ref[...] = jnp.maximum(y, 0.0)


def kernel(x, edge_index, W, b):
    ei = edge_index.astype(jnp.int32)
    src2d = ei[0].reshape(NW, NCHUNK, C)
    dst2d = ei[1].reshape(NW, NCHUNK, C)
    z128 = jnp.zeros((OROWS, D), jnp.float32)
    z16 = jnp.zeros((OROWS, DG), jnp.float32)
    ones16 = jnp.ones((C, DG), jnp.float32)
    partials, degs = _sc_aggregate(x, src2d, dst2d, z128, z16, ones16)

    BN = 1000
    out = pl.pallas_call(
        _tc_finish,
        grid=(N // BN,),
        in_specs=[
            pl.BlockSpec((NC, BN, D), lambda i: (0, i, 0)),
            pl.BlockSpec((NC, BN, DG), lambda i: (0, i, 0)),
            pl.BlockSpec((BN, D), lambda i: (i, 0)),
            pl.BlockSpec((D, D), lambda i: (0, 0)),
            pl.BlockSpec((1, D), lambda i: (0, 0)),
        ],
        out_specs=pl.BlockSpec((BN, D), lambda i: (i, 0)),
        out_shape=jax.ShapeDtypeStruct((N, D), jnp.float32),
    )(partials, degs, x, W, b.reshape(1, D))
    return out


# speedup vs baseline: 1.5429x; 1.0229x over previous
import functools

import jax
import jax.numpy as jnp
from jax import lax
from jax.experimental import pallas as pl
from jax.experimental.pallas import tpu as pltpu
from jax.experimental.pallas import tpu_sc as plsc

N = 10000
E = 320000
D = 128
DG = 16
NC = 2
NS = 16
NW = NC * NS
C = 80
NCHUNK = E // (NW * C)
ROWS_PER_TILE = N // NS
OROWS = 125
IH = 63
QD = 32
NBUF = 3

_mesh = plsc.VectorSubcoreMesh(
    core_axis_name="c", subcore_axis_name="s", num_cores=NC, num_subcores=NS
)


@functools.partial(
    pl.kernel,
    out_type=(
        jax.ShapeDtypeStruct((NC, N, D), jnp.float32),
        jax.ShapeDtypeStruct((NC, N, DG), jnp.float32),
    ),
    mesh=_mesh,
    scratch_types=[
        pltpu.VMEM((IH, C), jnp.int32),
        pltpu.VMEM((QD, C), jnp.int32),
        pltpu.VMEM((NBUF, C, D), jnp.float32),
        pltpu.VMEM((C, DG), jnp.float32),
        pltpu.VMEM_SHARED((N, D), jnp.float32),
        pltpu.VMEM_SHARED((N, DG), jnp.float32),
        pltpu.SemaphoreType.DMA((NBUF,)),
        pltpu.SemaphoreType.DMA((NBUF,)),
        pltpu.SemaphoreType.DMA((NBUF,)),
    ],
    compiler_params=pltpu.CompilerParams(use_tc_tiling_on_sc=False),
)
def _sc_aggregate(x_hbm, src_hbm, dst_hbm, z128_hbm, z16_hbm, ones_hbm,
                  out_hbm, deg_hbm,
                  src_v, dst_v, rows_v, ones_v, acc, dacc,
                  sem, sem_s, sem_d):
    cid = lax.axis_index("c")
    sid = lax.axis_index("s")
    wid = cid * NS + sid

    for kz in range(ROWS_PER_TILE // OROWS):
        r0 = sid * ROWS_PER_TILE + kz * OROWS
        pltpu.async_copy(z128_hbm, acc.at[pl.ds(r0, OROWS)], sem_s.at[0])
        pltpu.async_copy(z16_hbm, dacc.at[pl.ds(r0, OROWS)], sem_s.at[1])
    pltpu.async_copy(ones_hbm, ones_v, sem_s.at[2])

    pltpu.sync_copy(src_hbm.at[wid].at[pl.ds(0, IH)], src_v)
    pltpu.sync_copy(dst_hbm.at[wid].at[pl.ds(0, QD)], dst_v)

    def _gather(j, buf):
        row = jnp.where(j < IH, j, j - IH)
        pltpu.async_copy(x_hbm.at[src_v.at[row]], rows_v.at[buf], sem.at[buf])

    def _gather_wait(buf):
        pltpu.make_async_copy(
            x_hbm.at[src_v.at[0]], rows_v.at[buf], sem.at[buf]).wait()

    def _scatter_wait(buf):
        pltpu.make_async_copy(
            rows_v.at[buf], acc.at[dst_v.at[0]], sem_s.at[buf]).wait()

    def _dscatter_wait(buf):
        pltpu.make_async_copy(
            ones_v, dacc.at[dst_v.at[0]], sem_d.at[buf]).wait()

    _gather(0, 0)
    _gather(1, 1)
    for kz in range(ROWS_PER_TILE // OROWS):
        r0 = sid * ROWS_PER_TILE + kz * OROWS
        pltpu.make_async_copy(z128_hbm, acc.at[pl.ds(r0, OROWS)], sem_s.at[0]).wait()
        pltpu.make_async_copy(z16_hbm, dacc.at[pl.ds(r0, OROWS)], sem_s.at[1]).wait()
    pltpu.make_async_copy(ones_hbm, ones_v, sem_s.at[2]).wait()
    plsc.subcore_barrier()


    def _edge_chunk(j, _):
        buf = lax.rem(j, NBUF)
        _gather_wait(buf)

        @pl.when(j >= 1)
        def _():
            _scatter_wait(lax.rem(j - 1, NBUF))
            _dscatter_wait(lax.rem(j - 1, NBUF))

        @pl.when(j == IH - 2)
        def _():
            pltpu.sync_copy(src_hbm.at[wid].at[pl.ds(IH, NCHUNK - IH)],
                            src_v.at[pl.ds(0, NCHUNK - IH)])

        for jq, qn in ((QD, QD), (2 * QD, QD), (3 * QD, NCHUNK - 3 * QD)):
            @pl.when(j == jq)
            def _(jq=jq, qn=qn):
                pltpu.sync_copy(dst_hbm.at[wid].at[pl.ds(jq, qn)],
                                dst_v.at[pl.ds(0, qn)])

        @pl.when(j + 2 < NCHUNK)
        def _():
            _gather(j + 2, lax.rem(j + 2, NBUF))

        drow = lax.rem(j, QD)
        pltpu.async_copy(rows_v.at[buf], acc.at[dst_v.at[drow]],
                         sem_s.at[buf], add=True)
        pltpu.async_copy(ones_v, dacc.at[dst_v.at[drow]],
                         sem_d.at[buf], add=True)
        return 0
    lax.fori_loop(0, NCHUNK, _edge_chunk, 0)
    _scatter_wait((NCHUNK - 1) % NBUF)
    _dscatter_wait((NCHUNK - 1) % NBUF)

    plsc.subcore_barrier()
    for kz in range(ROWS_PER_TILE // OROWS):
        r0 = sid * ROWS_PER_TILE + kz * OROWS
        pltpu.async_copy(acc.at[pl.ds(r0, OROWS)],
                         out_hbm.at[cid].at[pl.ds(r0, OROWS)], sem.at[0])
    pltpu.async_copy(dacc.at[pl.ds(sid * ROWS_PER_TILE, ROWS_PER_TILE)],
                     deg_hbm.at[cid].at[pl.ds(sid * ROWS_PER_TILE, ROWS_PER_TILE)],
                     sem.at[1])
    for kz in range(ROWS_PER_TILE // OROWS):
        r0 = sid * ROWS_PER_TILE + kz * OROWS
        pltpu.make_async_copy(acc.at[pl.ds(r0, OROWS)],
                              out_hbm.at[cid].at[pl.ds(r0, OROWS)], sem.at[0]).wait()
    pltpu.make_async_copy(dacc.at[pl.ds(sid * ROWS_PER_TILE, ROWS_PER_TILE)],
                          deg_hbm.at[cid].at[pl.ds(sid * ROWS_PER_TILE, ROWS_PER_TILE)],
                          sem.at[1]).wait()


def _tc_finish(p_ref, d_ref, x_ref, w_ref, b_ref, o_ref):
    agg = p_ref[0] + p_ref[1]
    deg16 = d_ref[0] + d_ref[1]
    deg = jnp.maximum(deg16[:, 0:1], 1.0)
    h = agg / deg + x_ref[...]
    y = jnp.dot(h, w_ref[...], preferred_element_type=jnp.float32) + b_ref[...]
    o_ref[...] = jnp.maximum(y, 0.0)


def kernel(x, edge_index, W, b):
    ei = edge_index.astype(jnp.int32)
    src2d = ei[0].reshape(NW, NCHUNK, C)
    dst2d = ei[1].reshape(NW, NCHUNK, C)
    z128 = jnp.zeros((OROWS, D), jnp.float32)
    z16 = jnp.zeros((OROWS, DG), jnp.float32)
    ones16 = jnp.ones((C, DG), jnp.float32)
    partials, degs = _sc_aggregate(x, src2d, dst2d, z128, z16, ones16)

    BN = 1000
    out = pl.pallas_call(
        _tc_finish,
        grid=(N // BN,),
        in_specs=[
            pl.BlockSpec((NC, BN, D), lambda i: (0, i, 0)),
            pl.BlockSpec((NC, BN, DG), lambda i: (0, i, 0)),
            pl.BlockSpec((BN, D), lambda i: (i, 0)),
            pl.BlockSpec((D, D), lambda i: (0, 0)),
            pl.BlockSpec((1, D), lambda i: (0, 0)),
        ],
        out_specs=pl.BlockSpec((BN, D), lambda i: (i, 0)),
        out_shape=jax.ShapeDtypeStruct((N, D), jnp.float32),
    )(partials, degs, x, W, b.reshape(1, D))
    return out


# speedup vs baseline: 1.6432x; 1.0650x over previous
import functools

import jax
import jax.numpy as jnp
from jax import lax
from jax.experimental import pallas as pl
from jax.experimental.pallas import tpu as pltpu
from jax.experimental.pallas import tpu_sc as plsc

N = 10000
E = 320000
D = 128
DG = 16
NC = 2
NS = 16
NW = NC * NS
C = 80
NCHUNK = E // (NW * C)
ROWS_PER_TILE = N // NS
OROWS = 125
IH = 63
QD = 32
NBUF = 3

_mesh = plsc.VectorSubcoreMesh(
    core_axis_name="c", subcore_axis_name="s", num_cores=NC, num_subcores=NS
)


@functools.partial(
    pl.kernel,
    out_type=(
        jax.ShapeDtypeStruct((NC, N, D), jnp.float32),
        jax.ShapeDtypeStruct((NC, N, DG), jnp.float32),
    ),
    mesh=_mesh,
    scratch_types=[
        pltpu.VMEM((IH, C), jnp.int32),
        pltpu.VMEM((QD, C), jnp.int32),
        pltpu.VMEM((NBUF, C, D), jnp.float32),
        pltpu.VMEM((C, DG), jnp.float32),
        pltpu.VMEM_SHARED((N, D), jnp.float32),
        pltpu.VMEM_SHARED((N, DG), jnp.float32),
        pltpu.SemaphoreType.DMA((NBUF,)),
        pltpu.SemaphoreType.DMA((NBUF,)),
        pltpu.SemaphoreType.DMA((NBUF,)),
    ],
    compiler_params=pltpu.CompilerParams(use_tc_tiling_on_sc=False),
)
def _sc_aggregate(x_hbm, edge_hbm, z128_hbm, z16_hbm, ones_hbm,
                  out_hbm, deg_hbm,
                  src_v, dst_v, rows_v, ones_v, acc, dacc,
                  sem, sem_s, sem_d):
    cid = lax.axis_index("c")
    sid = lax.axis_index("s")
    wid = cid * NS + sid
    src_hbm = edge_hbm.at[0]
    dst_hbm = edge_hbm.at[1]

    for kz in range(ROWS_PER_TILE // OROWS):
        r0 = sid * ROWS_PER_TILE + kz * OROWS
        pltpu.async_copy(z128_hbm, acc.at[pl.ds(r0, OROWS)], sem_s.at[0])
        pltpu.async_copy(z16_hbm, dacc.at[pl.ds(r0, OROWS)], sem_s.at[1])
    pltpu.async_copy(ones_hbm, ones_v, sem_s.at[2])

    pltpu.sync_copy(src_hbm.at[wid].at[pl.ds(0, IH)], src_v)
    pltpu.sync_copy(dst_hbm.at[wid].at[pl.ds(0, QD)], dst_v)

    def _gather(j, buf):
        row = jnp.where(j < IH, j, j - IH)
        pltpu.async_copy(x_hbm.at[src_v.at[row]], rows_v.at[buf], sem.at[buf])

    def _gather_wait(buf):
        pltpu.make_async_copy(
            x_hbm.at[src_v.at[0]], rows_v.at[buf], sem.at[buf]).wait()

    def _scatter_wait(buf):
        pltpu.make_async_copy(
            rows_v.at[buf], acc.at[dst_v.at[0]], sem_s.at[buf]).wait()

    def _dscatter_wait(buf):
        pltpu.make_async_copy(
            ones_v, dacc.at[dst_v.at[0]], sem_d.at[buf]).wait()

    _gather(0, 0)
    _gather(1, 1)
    for kz in range(ROWS_PER_TILE // OROWS):
        r0 = sid * ROWS_PER_TILE + kz * OROWS
        pltpu.make_async_copy(z128_hbm, acc.at[pl.ds(r0, OROWS)], sem_s.at[0]).wait()
        pltpu.make_async_copy(z16_hbm, dacc.at[pl.ds(r0, OROWS)], sem_s.at[1]).wait()
    pltpu.make_async_copy(ones_hbm, ones_v, sem_s.at[2]).wait()
    plsc.subcore_barrier()


    def _edge_chunk(j, _):
        buf = lax.rem(j, NBUF)
        _gather_wait(buf)

        @pl.when(j >= 1)
        def _():
            _scatter_wait(lax.rem(j - 1, NBUF))
            _dscatter_wait(lax.rem(j - 1, NBUF))

        @pl.when(j == IH - 2)
        def _():
            pltpu.sync_copy(src_hbm.at[wid].at[pl.ds(IH, NCHUNK - IH)],
                            src_v.at[pl.ds(0, NCHUNK - IH)])

        for jq, qn in ((QD, QD), (2 * QD, QD), (3 * QD, NCHUNK - 3 * QD)):
            @pl.when(j == jq)
            def _(jq=jq, qn=qn):
                pltpu.sync_copy(dst_hbm.at[wid].at[pl.ds(jq, qn)],
                                dst_v.at[pl.ds(0, qn)])

        @pl.when(j + 2 < NCHUNK)
        def _():
            _gather(j + 2, lax.rem(j + 2, NBUF))

        drow = lax.rem(j, QD)
        pltpu.async_copy(rows_v.at[buf], acc.at[dst_v.at[drow]],
                         sem_s.at[buf], add=True)
        pltpu.async_copy(ones_v, dacc.at[dst_v.at[drow]],
                         sem_d.at[buf], add=True)
        return 0
    lax.fori_loop(0, NCHUNK, _edge_chunk, 0)
    _scatter_wait((NCHUNK - 1) % NBUF)
    _dscatter_wait((NCHUNK - 1) % NBUF)

    plsc.subcore_barrier()
    for kz in range(ROWS_PER_TILE // OROWS):
        r0 = sid * ROWS_PER_TILE + kz * OROWS
        pltpu.async_copy(acc.at[pl.ds(r0, OROWS)],
                         out_hbm.at[cid].at[pl.ds(r0, OROWS)], sem.at[0])
    pltpu.async_copy(dacc.at[pl.ds(sid * ROWS_PER_TILE, ROWS_PER_TILE)],
                     deg_hbm.at[cid].at[pl.ds(sid * ROWS_PER_TILE, ROWS_PER_TILE)],
                     sem.at[1])
    for kz in range(ROWS_PER_TILE // OROWS):
        r0 = sid * ROWS_PER_TILE + kz * OROWS
        pltpu.make_async_copy(acc.at[pl.ds(r0, OROWS)],
                              out_hbm.at[cid].at[pl.ds(r0, OROWS)], sem.at[0]).wait()
    pltpu.make_async_copy(dacc.at[pl.ds(sid * ROWS_PER_TILE, ROWS_PER_TILE)],
                          deg_hbm.at[cid].at[pl.ds(sid * ROWS_PER_TILE, ROWS_PER_TILE)],
                          sem.at[1]).wait()


def _tc_finish(p_ref, d_ref, x_ref, w_ref, b_ref, o_ref):
    agg = p_ref[0] + p_ref[1]
    deg16 = d_ref[0] + d_ref[1]
    deg = jnp.maximum(deg16[:, 0:1], 1.0)
    h = agg / deg + x_ref[...]
    y = jnp.dot(h, w_ref[...], preferred_element_type=jnp.float32) + b_ref[...]
    o_ref[...] = jnp.maximum(y, 0.0)


def kernel(x, edge_index, W, b):
    ei = edge_index.astype(jnp.int32).reshape(2, NW, NCHUNK, C)
    z128 = jnp.zeros((OROWS, D), jnp.float32)
    z16 = jnp.zeros((OROWS, DG), jnp.float32)
    ones16 = jnp.ones((C, DG), jnp.float32)
    partials, degs = _sc_aggregate(x, ei, z128, z16, ones16)

    BN = 1000
    out = pl.pallas_call(
        _tc_finish,
        grid=(N // BN,),
        in_specs=[
            pl.BlockSpec((NC, BN, D), lambda i: (0, i, 0)),
            pl.BlockSpec((NC, BN, DG), lambda i: (0, i, 0)),
            pl.BlockSpec((BN, D), lambda i: (i, 0)),
            pl.BlockSpec((D, D), lambda i: (0, 0)),
            pl.BlockSpec((1, D), lambda i: (0, 0)),
        ],
        out_specs=pl.BlockSpec((BN, D), lambda i: (i, 0)),
        out_shape=jax.ShapeDtypeStruct((N, D), jnp.float32),
    )(partials, degs, x, W, b.reshape(1, D))
    return out
